# baseline (speedup 1.0000x reference)
import functools

import jax
import jax.numpy as jnp
from jax import lax
from jax.experimental import pallas as pl
from jax.experimental.pallas import tpu as pltpu
from jax.experimental.pallas import tpu_sc as plsc

N_INTERVAL = 100
K_KEYPOINTS = 100000
B = 16384
D = 3

NC = 2
NS = 16
L = 16
NW = NC * NS
BPW = B // NW
G = 128
NG = BPW // G
NR = 2 * D
SPB = G // L

_mesh = plsc.VectorSubcoreMesh(
    core_axis_name="c", subcore_axis_name="s", num_cores=NC, num_subcores=NS
)


@functools.partial(
    pl.kernel,
    out_type=jax.ShapeDtypeStruct((D * B,), jnp.float32),
    mesh=_mesh,
    compiler_params=pltpu.CompilerParams(
        needs_layout_passes=False, use_tc_tiling_on_sc=False,
        skip_device_barrier=True, disable_bounds_checks=True,
        disable_semaphore_checks=True),
    scratch_types=[
        pltpu.VMEM((BPW,), jnp.int32),
        pltpu.VMEM((D, BPW), jnp.float32),
        pltpu.VMEM((BPW,), jnp.float32),
        pltpu.VMEM((BPW,), jnp.float32),
        pltpu.VMEM((NR, BPW), jnp.int32),
        pltpu.VMEM((NR, BPW), jnp.float32),
        pltpu.VMEM((D, BPW), jnp.float32),
        pltpu.SemaphoreType.DMA,
        pltpu.SemaphoreType.DMA,
    ],
)
def _position_sc(x_hbm, i_hbm, delta_hbm, out_hbm,
                 i_v, x_v, wl_v, wr_v, idx_v, d_v, out_v, sem_x, sem_g):
    wid = lax.axis_index("s") * NC + lax.axis_index("c")
    base = wid * BPW

    pltpu.sync_copy(i_hbm.at[pl.ds(base, BPW)], i_v)
    x_cps = [
        pltpu.async_copy(x_hbm.at[pl.ds(c * B + base, BPW)], x_v.at[c], sem_x)
        for c in range(D)
    ]

    vinv = jnp.full((L,), 1.0 / N_INTERVAL, jnp.float32)
    vmax_i = jnp.full((L,), K_KEYPOINTS - 2, jnp.int32)
    v1_f = jnp.full((L,), 1.0, jnp.float32)
    offs = [jnp.full((L,), c * K_KEYPOINTS + s, jnp.int32)
            for c in range(D) for s in range(2)]
    for s in range(BPW // L):
        iv = i_v[pl.ds(s * L, L)]
        raw = iv.astype(jnp.float32) * vinv
        left = jnp.minimum(raw.astype(jnp.int32), vmax_i)
        leftf = left.astype(jnp.float32)
        wl_v[pl.ds(s * L, L)] = leftf + v1_f - raw
        wr_v[pl.ds(s * L, L)] = raw - leftf
        for r in range(NR):
            idx_v[r, pl.ds(s * L, L)] = left + offs[r]
    g_cps = [
        pltpu.async_copy(delta_hbm.at[idx_v.at[r]], d_v.at[r], sem_g)
        for r in range(NR)
    ]
    for cp in x_cps:
        cp.wait()
    for cp in g_cps:
        cp.wait()

    for s in range(BPW // L):
        wl = wl_v[pl.ds(s * L, L)]
        wr = wr_v[pl.ds(s * L, L)]
        for c in range(D):
            dl = d_v[2 * c, pl.ds(s * L, L)]
            dr = d_v[2 * c + 1, pl.ds(s * L, L)]
            xc = x_v[c, pl.ds(s * L, L)]
            out_v[c, pl.ds(s * L, L)] = xc + wl * dl + wr * dr

    o_cps = [
        pltpu.async_copy(out_v.at[c], out_hbm.at[pl.ds(c * B + base, BPW)],
                         sem_x)
        for c in range(D)
    ]
    for cp in o_cps:
        cp.wait()


def kernel(x, i, delta):
    out_flat = _position_sc(
        x.T.reshape(-1), i, delta.T.reshape(-1))
    return out_flat.reshape(D, B).T


# speedup vs baseline: 1.0097x; 1.0097x over previous
import functools

import jax
import jax.numpy as jnp
from jax import lax
from jax.experimental import pallas as pl
from jax.experimental.pallas import tpu as pltpu
from jax.experimental.pallas import tpu_sc as plsc

N_INTERVAL = 100
K_KEYPOINTS = 100000
B = 16384
D = 3

NC = 2
NS = 16
L = 16
NW = NC * NS
BPW = B // NW
G = 128
NG = BPW // G
NR = 2 * D
SPB = G // L

_mesh = plsc.VectorSubcoreMesh(
    core_axis_name="c", subcore_axis_name="s", num_cores=NC, num_subcores=NS
)


@functools.partial(
    pl.kernel,
    out_type=jax.ShapeDtypeStruct((D * B,), jnp.float32),
    mesh=_mesh,
    compiler_params=pltpu.CompilerParams(
        needs_layout_passes=False, use_tc_tiling_on_sc=False,
        skip_device_barrier=True, disable_bounds_checks=True,
        disable_semaphore_checks=True),
    scratch_types=[
        pltpu.VMEM((BPW,), jnp.int32),
        pltpu.VMEM((D, BPW), jnp.float32),
        pltpu.VMEM((BPW,), jnp.float32),
        pltpu.VMEM((BPW,), jnp.float32),
        pltpu.VMEM((NR * NG, G), jnp.int32),
        pltpu.VMEM((NR * NG, G), jnp.float32),
        pltpu.VMEM((D, BPW), jnp.float32),
        pltpu.SemaphoreType.DMA,
        pltpu.SemaphoreType.DMA,
        pltpu.SemaphoreType.DMA,
        pltpu.SemaphoreType.DMA,
        pltpu.SemaphoreType.DMA,
        pltpu.SemaphoreType.DMA,
    ],
)
def _position_sc(x_hbm, i_hbm, delta_hbm, out_hbm,
                 i_v, x_v, wl_v, wr_v, idx_v, d_v, out_v,
                 sem_x, sem_o, sem_g0, sem_g1, sem_g2, sem_g3):
    sem_g = [sem_g0, sem_g1, sem_g2, sem_g3]
    wid = lax.axis_index("s") * NC + lax.axis_index("c")
    base = wid * BPW

    pltpu.sync_copy(i_hbm.at[pl.ds(base, BPW)], i_v)
    x_cps = [
        pltpu.async_copy(x_hbm.at[pl.ds(c * B + base, BPW)], x_v.at[c], sem_x)
        for c in range(D)
    ]

    vinv = jnp.full((L,), 1.0 / N_INTERVAL, jnp.float32)
    vmax_i = jnp.full((L,), K_KEYPOINTS - 2, jnp.int32)
    v1_f = jnp.full((L,), 1.0, jnp.float32)
    offs = [jnp.full((L,), c * K_KEYPOINTS + s, jnp.int32)
            for c in range(D) for s in range(2)]
    g_cps = [[] for _ in range(NG)]
    for blk in range(NG):
        for sub in range(SPB):
            s = blk * SPB + sub
            iv = i_v[pl.ds(s * L, L)]
            raw = iv.astype(jnp.float32) * vinv
            left = jnp.minimum(raw.astype(jnp.int32), vmax_i)
            leftf = left.astype(jnp.float32)
            wl_v[pl.ds(s * L, L)] = leftf + v1_f - raw
            wr_v[pl.ds(s * L, L)] = raw - leftf
            for r in range(NR):
                idx_v[r * NG + blk, pl.ds(sub * L, L)] = left + offs[r]
        for r in range(NR):
            row = r * NG + blk
            g_cps[blk].append(pltpu.async_copy(
                delta_hbm.at[idx_v.at[row]], d_v.at[row], sem_g[blk]))
    for cp in x_cps:
        cp.wait()

    o_cps = []
    for blk in range(NG):
        for cp in g_cps[blk]:
            cp.wait()
        for sub in range(SPB):
            s = blk * SPB + sub
            col = sub * L
            wl = wl_v[pl.ds(s * L, L)]
            wr = wr_v[pl.ds(s * L, L)]
            for c in range(D):
                dl = d_v[(2 * c) * NG + blk, pl.ds(col, L)]
                dr = d_v[(2 * c + 1) * NG + blk, pl.ds(col, L)]
                xc = x_v[c, pl.ds(s * L, L)]
                out_v[c, pl.ds(s * L, L)] = xc + wl * dl + wr * dr
        for c in range(D):
            o_cps.append(pltpu.async_copy(
                out_v.at[c, pl.ds(blk * G, G)],
                out_hbm.at[pl.ds(c * B + base + blk * G, G)], sem_o))
    for cp in o_cps:
        cp.wait()


def kernel(x, i, delta):
    out_flat = _position_sc(
        x.T.reshape(-1), i, delta.T.reshape(-1))
    return out_flat.reshape(D, B).T


# speedup vs baseline: 1.0227x; 1.0129x over previous
import functools

import jax
import jax.numpy as jnp
from jax import lax
from jax.experimental import pallas as pl
from jax.experimental.pallas import tpu as pltpu
from jax.experimental.pallas import tpu_sc as plsc

N_INTERVAL = 100
K_KEYPOINTS = 100000
B = 16384
D = 3

NC = 2
NS = 16
L = 16
NW = NC * NS
BPW = B // NW
G = 128
NG = BPW // G
NR = 2 * D
SPB = G // L

_mesh = plsc.VectorSubcoreMesh(
    core_axis_name="c", subcore_axis_name="s", num_cores=NC, num_subcores=NS
)


@functools.partial(
    pl.kernel,
    out_type=jax.ShapeDtypeStruct((D * B,), jnp.float32),
    mesh=_mesh,
    compiler_params=pltpu.CompilerParams(
        needs_layout_passes=False, use_tc_tiling_on_sc=False,
        skip_device_barrier=True, disable_bounds_checks=True,
        disable_semaphore_checks=True),
    scratch_types=[
        pltpu.VMEM((BPW,), jnp.int32),
        pltpu.VMEM((D, BPW), jnp.float32),
        pltpu.VMEM((BPW,), jnp.float32),
        pltpu.VMEM((BPW,), jnp.float32),
        pltpu.VMEM((NR * NG, G), jnp.int32),
        pltpu.VMEM((NR * NG, G), jnp.float32),
        pltpu.VMEM((D, BPW), jnp.float32),
        pltpu.SemaphoreType.DMA,
        pltpu.SemaphoreType.DMA,
        pltpu.SemaphoreType.DMA,
        pltpu.SemaphoreType.DMA,
        pltpu.SemaphoreType.DMA,
        pltpu.SemaphoreType.DMA,
    ],
)
def _position_sc(xd_hbm, i_hbm, out_hbm,
                 i_v, x_v, wl_v, wr_v, idx_v, d_v, out_v,
                 sem_x, sem_o, sem_g0, sem_g1, sem_g2, sem_g3):
    sem_g = [sem_g0, sem_g1, sem_g2, sem_g3]
    wid = lax.axis_index("s") * NC + lax.axis_index("c")
    base = wid * BPW

    pltpu.sync_copy(i_hbm.at[pl.ds(base, BPW)], i_v)
    x_cps = [
        pltpu.async_copy(xd_hbm.at[pl.ds(c * B + base, BPW)], x_v.at[c],
                         sem_x)
        for c in range(D)
    ]

    vinv = jnp.full((L,), 1.0 / N_INTERVAL, jnp.float32)
    vmax_i = jnp.full((L,), K_KEYPOINTS - 2, jnp.int32)
    v1_f = jnp.full((L,), 1.0, jnp.float32)
    offs = [jnp.full((L,), D * B + c * K_KEYPOINTS + s, jnp.int32)
            for c in range(D) for s in range(2)]
    g_cps = [[] for _ in range(NG)]
    for blk in range(NG):
        for sub in range(SPB):
            s = blk * SPB + sub
            iv = i_v[pl.ds(s * L, L)]
            raw = iv.astype(jnp.float32) * vinv
            left = jnp.minimum(raw.astype(jnp.int32), vmax_i)
            leftf = left.astype(jnp.float32)
            wl_v[pl.ds(s * L, L)] = leftf + v1_f - raw
            wr_v[pl.ds(s * L, L)] = raw - leftf
            for r in range(NR):
                idx_v[r * NG + blk, pl.ds(sub * L, L)] = left + offs[r]
        for r in range(NR):
            row = r * NG + blk
            g_cps[blk].append(pltpu.async_copy(
                xd_hbm.at[idx_v.at[row]], d_v.at[row], sem_g[blk]))
    for cp in x_cps:
        cp.wait()

    o_cps = []
    for blk in range(NG):
        for cp in g_cps[blk]:
            cp.wait()
        for sub in range(SPB):
            s = blk * SPB + sub
            col = sub * L
            wl = wl_v[pl.ds(s * L, L)]
            wr = wr_v[pl.ds(s * L, L)]
            for c in range(D):
                dl = d_v[(2 * c) * NG + blk, pl.ds(col, L)]
                dr = d_v[(2 * c + 1) * NG + blk, pl.ds(col, L)]
                xc = x_v[c, pl.ds(s * L, L)]
                out_v[c, pl.ds(s * L, L)] = xc + wl * dl + wr * dr
        for c in range(D):
            o_cps.append(pltpu.async_copy(
                out_v.at[c, pl.ds(blk * G, G)],
                out_hbm.at[pl.ds(c * B + base + blk * G, G)], sem_o))
    for cp in o_cps:
        cp.wait()


def kernel(x, i, delta):
    xd = jnp.concatenate([x.T.reshape(-1), delta.T.reshape(-1)])
    out_flat = _position_sc(xd, i)
    return out_flat.reshape(D, B).T


# speedup vs baseline: 1.0266x; 1.0037x over previous
import functools

import jax
import jax.numpy as jnp
from jax import lax
from jax.experimental import pallas as pl
from jax.experimental.pallas import tpu as pltpu
from jax.experimental.pallas import tpu_sc as plsc

N_INTERVAL = 100
K_KEYPOINTS = 100000
B = 16384
D = 3

NC = 2
NS = 16
L = 16
NW = NC * NS
BPW = B // NW
G = 128
NG = BPW // G
NR = 2 * D
SPB = G // L

_mesh = plsc.VectorSubcoreMesh(
    core_axis_name="c", subcore_axis_name="s", num_cores=NC, num_subcores=NS
)


@functools.partial(
    pl.kernel,
    out_type=jax.ShapeDtypeStruct((D * B,), jnp.float32),
    mesh=_mesh,
    compiler_params=pltpu.CompilerParams(
        needs_layout_passes=False, use_tc_tiling_on_sc=False,
        skip_device_barrier=True, disable_bounds_checks=True,
        disable_semaphore_checks=True),
    scratch_types=[
        pltpu.VMEM((BPW,), jnp.int32),
        pltpu.VMEM((D, BPW), jnp.float32),
        pltpu.VMEM((BPW,), jnp.float32),
        pltpu.VMEM((BPW,), jnp.float32),
        pltpu.VMEM((NR * NG, G), jnp.int32),
        pltpu.VMEM((NR * NG, G), jnp.float32),
        pltpu.VMEM((D, BPW), jnp.float32),
        pltpu.SemaphoreType.DMA,
        pltpu.SemaphoreType.DMA,
        pltpu.SemaphoreType.DMA,
        pltpu.SemaphoreType.DMA,
        pltpu.SemaphoreType.DMA,
        pltpu.SemaphoreType.DMA,
        pltpu.SemaphoreType.DMA,
    ],
)
def _position_sc(xd_hbm, i_hbm, out_hbm,
                 i_v, x_v, wl_v, wr_v, idx_v, d_v, out_v,
                 sem_i, sem_x, sem_o, sem_g0, sem_g1, sem_g2, sem_g3):
    sem_g = [sem_g0, sem_g1, sem_g2, sem_g3]
    wid = lax.axis_index("s") * NC + lax.axis_index("c")
    base = wid * BPW

    i_cps = [
        pltpu.async_copy(i_hbm.at[pl.ds(base + blk * G, G)],
                         i_v.at[pl.ds(blk * G, G)], sem_i)
        for blk in range(NG)
    ]
    x_cps = [
        pltpu.async_copy(xd_hbm.at[pl.ds(c * B + base, BPW)], x_v.at[c],
                         sem_x)
        for c in range(D)
    ]

    vinv = jnp.full((L,), 1.0 / N_INTERVAL, jnp.float32)
    vmax_i = jnp.full((L,), K_KEYPOINTS - 2, jnp.int32)
    v1_f = jnp.full((L,), 1.0, jnp.float32)
    offs = [jnp.full((L,), D * B + c * K_KEYPOINTS + s, jnp.int32)
            for c in range(D) for s in range(2)]
    g_cps = [[] for _ in range(NG)]
    for blk in range(NG):
        i_cps[blk].wait()
        for sub in range(SPB):
            s = blk * SPB + sub
            iv = i_v[pl.ds(s * L, L)]
            raw = iv.astype(jnp.float32) * vinv
            left = jnp.minimum(raw.astype(jnp.int32), vmax_i)
            leftf = left.astype(jnp.float32)
            wl_v[pl.ds(s * L, L)] = leftf + v1_f - raw
            wr_v[pl.ds(s * L, L)] = raw - leftf
            for r in range(NR):
                idx_v[r * NG + blk, pl.ds(sub * L, L)] = left + offs[r]
        for r in range(NR):
            row = r * NG + blk
            g_cps[blk].append(pltpu.async_copy(
                xd_hbm.at[idx_v.at[row]], d_v.at[row], sem_g[blk]))
    for cp in x_cps:
        cp.wait()

    o_cps = []
    for blk in range(NG):
        for cp in g_cps[blk]:
            cp.wait()
        for sub in range(SPB):
            s = blk * SPB + sub
            col = sub * L
            wl = wl_v[pl.ds(s * L, L)]
            wr = wr_v[pl.ds(s * L, L)]
            for c in range(D):
                dl = d_v[(2 * c) * NG + blk, pl.ds(col, L)]
                dr = d_v[(2 * c + 1) * NG + blk, pl.ds(col, L)]
                xc = x_v[c, pl.ds(s * L, L)]
                out_v[c, pl.ds(s * L, L)] = xc + wl * dl + wr * dr
        for c in range(D):
            o_cps.append(pltpu.async_copy(
                out_v.at[c, pl.ds(blk * G, G)],
                out_hbm.at[pl.ds(c * B + base + blk * G, G)], sem_o))
    for cp in o_cps:
        cp.wait()


def kernel(x, i, delta):
    xd = jnp.concatenate([x.T.reshape(-1), delta.T.reshape(-1)])
    out_flat = _position_sc(xd, i)
    return out_flat.reshape(D, B).T


# speedup vs baseline: 1.0317x; 1.0050x over previous
import functools

import jax
import jax.numpy as jnp
from jax import lax
from jax.experimental import pallas as pl
from jax.experimental.pallas import tpu as pltpu
from jax.experimental.pallas import tpu_sc as plsc

N_INTERVAL = 100
K_KEYPOINTS = 100000
B = 16384
D = 3

NC = 2
NS = 16
L = 16
NW = NC * NS
BPW = B // NW
G = 128
NG = BPW // G
NR = 2 * D
SPB = G // L

_mesh = plsc.VectorSubcoreMesh(
    core_axis_name="c", subcore_axis_name="s", num_cores=NC, num_subcores=NS
)


@functools.partial(
    pl.kernel,
    out_type=jax.ShapeDtypeStruct((D * B,), jnp.float32),
    mesh=_mesh,
    compiler_params=pltpu.CompilerParams(
        needs_layout_passes=False, use_tc_tiling_on_sc=False,
        skip_device_barrier=True, disable_bounds_checks=True,
        disable_semaphore_checks=True),
    scratch_types=[
        pltpu.VMEM((BPW,), jnp.int32),
        pltpu.VMEM((D, BPW), jnp.float32),
        pltpu.VMEM((BPW,), jnp.float32),
        pltpu.VMEM((BPW,), jnp.float32),
        pltpu.VMEM((NR * NG * G,), jnp.int32),
        pltpu.VMEM((NR * NG * G,), jnp.float32),
        pltpu.VMEM((D, BPW), jnp.float32),
        pltpu.SemaphoreType.DMA,
        pltpu.SemaphoreType.DMA,
        pltpu.SemaphoreType.DMA,
        pltpu.SemaphoreType.DMA,
        pltpu.SemaphoreType.DMA,
        pltpu.SemaphoreType.DMA,
        pltpu.SemaphoreType.DMA,
    ],
)
def _position_sc(xd_hbm, i_hbm, out_hbm,
                 i_v, x_v, wl_v, wr_v, idx_v, d_v, out_v,
                 sem_i, sem_x, sem_o, sem_g0, sem_g1, sem_g2, sem_g3):
    sem_g = [sem_g0, sem_g1, sem_g2, sem_g3]
    wid = lax.axis_index("s") * NC + lax.axis_index("c")
    base = wid * BPW

    i_cps = [
        pltpu.async_copy(i_hbm.at[pl.ds(base + blk * G, G)],
                         i_v.at[pl.ds(blk * G, G)], sem_i)
        for blk in range(NG)
    ]
    x_cps = [
        pltpu.async_copy(xd_hbm.at[pl.ds(c * B + base, BPW)], x_v.at[c],
                         sem_x)
        for c in range(D)
    ]

    vinv = jnp.full((L,), 1.0 / N_INTERVAL, jnp.float32)
    vmax_i = jnp.full((L,), K_KEYPOINTS - 2, jnp.int32)
    v1_f = jnp.full((L,), 1.0, jnp.float32)
    offs = [jnp.full((L,), D * B + c * K_KEYPOINTS + s, jnp.int32)
            for c in range(D) for s in range(2)]
    g_cps = [[] for _ in range(NG)]
    for blk in range(NG):
        i_cps[blk].wait()
        for sub in range(SPB):
            s = blk * SPB + sub
            iv = i_v[pl.ds(s * L, L)]
            raw = iv.astype(jnp.float32) * vinv
            left = jnp.minimum(raw.astype(jnp.int32), vmax_i)
            leftf = left.astype(jnp.float32)
            wl_v[pl.ds(s * L, L)] = leftf + v1_f - raw
            wr_v[pl.ds(s * L, L)] = raw - leftf
            for r in range(NR):
                idx_v[pl.ds((blk * NR + r) * G + sub * L, L)] = left + offs[r]
        g_cps[blk].append(pltpu.async_copy(
            xd_hbm.at[idx_v.at[pl.ds(blk * NR * G, NR * G)]],
            d_v.at[pl.ds(blk * NR * G, NR * G)], sem_g[blk]))
    for cp in x_cps:
        cp.wait()

    o_cps = []
    for blk in range(NG):
        for cp in g_cps[blk]:
            cp.wait()
        for sub in range(SPB):
            s = blk * SPB + sub
            col = sub * L
            wl = wl_v[pl.ds(s * L, L)]
            wr = wr_v[pl.ds(s * L, L)]
            for c in range(D):
                dl = d_v[pl.ds((blk * NR + 2 * c) * G + col, L)]
                dr = d_v[pl.ds((blk * NR + 2 * c + 1) * G + col, L)]
                xc = x_v[c, pl.ds(s * L, L)]
                out_v[c, pl.ds(s * L, L)] = xc + wl * dl + wr * dr
        for c in range(D):
            o_cps.append(pltpu.async_copy(
                out_v.at[c, pl.ds(blk * G, G)],
                out_hbm.at[pl.ds(c * B + base + blk * G, G)], sem_o))
    for cp in o_cps:
        cp.wait()


def kernel(x, i, delta):
    xd = jnp.concatenate([x.T.reshape(-1), delta.T.reshape(-1)])
    out_flat = _position_sc(xd, i)
    return out_flat.reshape(D, B).T
